# Initial kernel scaffold; baseline (speedup 1.0000x reference)
import jax
import jax.numpy as jnp
from jax.experimental import pallas as pl


def kernel(x, edge_index):
    raise NotImplementedError("write your pallas kernel here")


# speedup vs baseline: 4.9357x; 4.9357x over previous
import functools

import jax
import jax.numpy as jnp
from jax import lax
from jax.experimental import pallas as pl
from jax.experimental.pallas import tpu as pltpu
from jax.experimental.pallas import tpu_sc as plsc

NC = 2
NS = 16
NW = NC * NS
CHUNK = 128


@functools.lru_cache(maxsize=None)
def _sc_partial_sums(n_nodes: int, d: int, n_chunks: int):
    acc_rows = -(-(n_nodes + 1) // (NS * CHUNK)) * (NS * CHUNK)
    zero_chunks_per_tile = acc_rows // NS // CHUNK
    out_rows_per_tile = acc_rows // NS
    assert d % 16 == 0

    mesh = plsc.VectorSubcoreMesh(core_axis_name="c", subcore_axis_name="s")

    @functools.partial(
        pl.kernel,
        out_type=jax.ShapeDtypeStruct((NC, acc_rows, d), jnp.float32),
        mesh=mesh,
        scratch_types=[
            pltpu.VMEM((n_chunks, CHUNK), jnp.int32),
            pltpu.VMEM((n_chunks, CHUNK), jnp.int32),
            pltpu.VMEM((CHUNK, d), jnp.float32),
            pltpu.VMEM_SHARED((acc_rows, d), jnp.float32),
            pltpu.SemaphoreType.DMA,
        ],
    )
    def sc_kernel(x_hbm, col_hbm, row_hbm, out_hbm, col_v, row_v, gbuf, acc, sem):
        cid = lax.axis_index("c")
        sid = lax.axis_index("s")
        wid = cid * NS + sid

        pltpu.sync_copy(col_hbm.at[wid], col_v)
        pltpu.sync_copy(row_hbm.at[wid], row_v)

        def zero_body(i, carry):
            for j in range(d // 16):
                gbuf[i, pl.ds(j * 16, 16)] = jnp.zeros((16,), jnp.float32)
            return carry
        lax.fori_loop(0, CHUNK, zero_body, 0)
        for k in range(zero_chunks_per_tile):
            pltpu.sync_copy(
                gbuf, acc.at[pl.ds((sid * zero_chunks_per_tile + k) * CHUNK, CHUNK)]
            )
        plsc.subcore_barrier()

        def body(j, carry):
            pltpu.async_copy(x_hbm.at[col_v.at[j]], gbuf, sem).wait()
            pltpu.sync_copy(gbuf, acc.at[row_v.at[j]], add=True)
            return carry
        lax.fori_loop(0, n_chunks, body, 0)
        plsc.subcore_barrier()

        pltpu.sync_copy(
            acc.at[pl.ds(sid * out_rows_per_tile, out_rows_per_tile)],
            out_hbm.at[cid, pl.ds(sid * out_rows_per_tile, out_rows_per_tile)],
        )

    return sc_kernel


@functools.lru_cache(maxsize=None)
def _combine(n_nodes: int, d: int):
    blk = 1000
    assert n_nodes % blk == 0

    def body(x_ref, a_ref, b_ref, o_ref):
        o_ref[:, :d] = x_ref[...]
        o_ref[:, d:] = a_ref[...] + b_ref[...]

    return pl.pallas_call(
        body,
        grid=(n_nodes // blk,),
        in_specs=[pl.BlockSpec((blk, d), lambda i: (i, 0))] * 3,
        out_specs=pl.BlockSpec((blk, 2 * d), lambda i: (i, 0)),
        out_shape=jax.ShapeDtypeStruct((n_nodes, 2 * d), jnp.float32),
    )


def kernel(x, edge_index):
    n_nodes, d = x.shape
    n_edges = edge_index.shape[1]
    ei = edge_index.astype(jnp.int32)
    row, col = ei[0], ei[1]

    per_round = NW * CHUNK
    n_chunks = -(-n_edges // per_round)
    e_pad = n_chunks * per_round
    if e_pad != n_edges:
        pad = e_pad - n_edges
        row = jnp.concatenate([row, jnp.full((pad,), n_nodes, jnp.int32)])
        col = jnp.concatenate([col, jnp.zeros((pad,), jnp.int32)])
    row3 = row.reshape(NW, n_chunks, CHUNK)
    col3 = col.reshape(NW, n_chunks, CHUNK)

    partial = _sc_partial_sums(n_nodes, d, n_chunks)(x, col3, row3)
    return _combine(n_nodes, d)(x, partial[0, :n_nodes], partial[1, :n_nodes])
